# baseline (speedup 1.0000x reference)
import functools

import jax
import jax.numpy as jnp
from jax import lax
from jax.experimental import pallas as pl
from jax.experimental.pallas import tpu as pltpu
from jax.experimental.pallas import tpu_sc as plsc

B, T, C, M = 16, 512, 256, 2048
ZROW = B * T
TPAD = B * T + 8
MM = M + 128

NC, NS = 2, 16
NW = NC * NS
ROWS = B * M
RPW = ROWS // NW
CH = 128
NCH = RPW // CH


def _idx_body(dur_ref, out_ref):
    b = pl.program_id(0)
    d = dur_ref[...].reshape(1, T)
    df = d.astype(jnp.float32)
    it = lax.broadcasted_iota(jnp.int32, (T, T), 0)
    jt = lax.broadcasted_iota(jnp.int32, (T, T), 1)
    tri = (it <= jt).astype(jnp.float32)
    csum = jnp.dot(df, tri, preferred_element_type=jnp.float32)
    csum_i = csum.astype(jnp.int32)
    mrow = lax.broadcasted_iota(jnp.int32, (M, T), 0)
    mask = csum_i <= mrow
    tid = jnp.sum(mask.astype(jnp.int32), axis=1, keepdims=True)
    gidx = jnp.where(tid >= T, ZROW, b * T + tid)
    total = jnp.sum(d)
    tot_col = jnp.broadcast_to(total, (MM - M, 1))
    out_ref[...] = jnp.concatenate([gidx, tot_col], axis=0).reshape(1, MM, 1)


_idx_call = pl.pallas_call(
    _idx_body,
    grid=(B,),
    in_specs=[pl.BlockSpec((1, 1, T), lambda b: (b, 0, 0))],
    out_specs=pl.BlockSpec((1, MM, 1), lambda b: (b, 0, 0)),
    out_shape=jax.ShapeDtypeStruct((B, MM, 1), jnp.int32),
)


@functools.cache
def _make_sc_gather():
    mesh = plsc.VectorSubcoreMesh(core_axis_name="c", subcore_axis_name="s")

    @functools.partial(
        pl.kernel,
        mesh=mesh,
        out_type=jax.ShapeDtypeStruct((ROWS, C), jnp.float32),
        scratch_types=[
            pltpu.VMEM((CH,), jnp.int32),
            pltpu.VMEM((CH, C), jnp.float32),
            pltpu.SemaphoreType.DMA,
        ],
    )
    def _sc_gather(table_hbm, gidx_hbm, out_hbm, idx_v, rows_v, sem):
        wid = lax.axis_index("s") * NC + lax.axis_index("c")
        base = wid * RPW
        for c in range(NCH):
            off = base + c * CH
            pltpu.sync_copy(gidx_hbm.at[pl.ds(off, CH)], idx_v)
            pltpu.async_copy(table_hbm.at[idx_v], rows_v, sem).wait()
            pltpu.sync_copy(rows_v, out_hbm.at[pl.ds(off, CH)])

    return _sc_gather


def kernel(x, duration, max_mel_len):
    table = jnp.concatenate(
        [x.reshape(B * T, C), jnp.zeros((TPAD - B * T, C), x.dtype)], axis=0)
    buf = _idx_call(duration.astype(jnp.int32).reshape(B, 1, T)).reshape(B, MM)
    gidx = buf[:, :M].reshape(ROWS)
    total = buf[:, M]
    out = _make_sc_gather()(table, gidx)
    return out.reshape(B, M, C), total


# speedup vs baseline: 1.0022x; 1.0022x over previous
import functools

import jax
import jax.numpy as jnp
from jax import lax
from jax.experimental import pallas as pl
from jax.experimental.pallas import tpu as pltpu
from jax.experimental.pallas import tpu_sc as plsc

B, T, C, M = 16, 512, 256, 2048
ZROW = B * T
TPAD = B * T + 8
MM = M + 128

NC, NS = 2, 16
NW = NC * NS
ROWS = B * M
RPW = ROWS // NW
CH = 128
NCH = RPW // CH


def _idx_body(dur_ref, out_ref):
    b = pl.program_id(0)
    d = dur_ref[...].reshape(1, T)
    df = d.astype(jnp.float32)
    it = lax.broadcasted_iota(jnp.int32, (T, T), 0)
    jt = lax.broadcasted_iota(jnp.int32, (T, T), 1)
    tri = (it <= jt).astype(jnp.float32)
    csum = jnp.dot(df, tri, preferred_element_type=jnp.float32)
    csum_i = csum.astype(jnp.int32)
    mrow = lax.broadcasted_iota(jnp.int32, (M, T), 0)
    mask = csum_i <= mrow
    tid = jnp.sum(mask.astype(jnp.int32), axis=1, keepdims=True)
    gidx = jnp.where(tid >= T, ZROW, b * T + tid)
    total = jnp.sum(d)
    tot_col = jnp.broadcast_to(total, (MM - M, 1))
    out_ref[...] = jnp.concatenate([gidx, tot_col], axis=0).reshape(1, MM, 1)


_idx_call = pl.pallas_call(
    _idx_body,
    grid=(B,),
    in_specs=[pl.BlockSpec((1, 1, T), lambda b: (b, 0, 0))],
    out_specs=pl.BlockSpec((1, MM, 1), lambda b: (b, 0, 0)),
    out_shape=jax.ShapeDtypeStruct((B, MM, 1), jnp.int32),
)


NB = 3


@functools.cache
def _make_sc_gather():
    mesh = plsc.VectorSubcoreMesh(core_axis_name="c", subcore_axis_name="s")

    @functools.partial(
        pl.kernel,
        mesh=mesh,
        out_type=jax.ShapeDtypeStruct((ROWS, C), jnp.float32),
        scratch_types=[
            pltpu.VMEM((RPW,), jnp.int32),
            pltpu.VMEM((NB, CH, C), jnp.float32),
            pltpu.SemaphoreType.DMA,
            pltpu.SemaphoreType.DMA,
            pltpu.SemaphoreType.DMA,
            pltpu.SemaphoreType.DMA,
            pltpu.SemaphoreType.DMA,
            pltpu.SemaphoreType.DMA,
        ],
    )
    def _sc_gather(table_hbm, gidx_hbm, out_hbm, idx_all, bufs, g0, g1, g2,
                   s0, s1, s2):
        gsem = (g0, g1, g2)
        ssem = (s0, s1, s2)
        wid = lax.axis_index("s") * NC + lax.axis_index("c")
        base = wid * RPW
        pltpu.sync_copy(gidx_hbm.at[pl.ds(base, RPW)], idx_all)

        def gather(c):
            j = c % NB
            return pltpu.async_copy(
                table_hbm.at[idx_all.at[pl.ds(c * CH, CH)]], bufs.at[j],
                gsem[j])

        def scatter(c):
            j = c % NB
            return pltpu.async_copy(
                bufs.at[j], out_hbm.at[pl.ds(base + c * CH, CH)], ssem[j])

        g = [None] * NCH
        s = [None] * NCH
        g[0] = gather(0)
        g[1] = gather(1)
        for c in range(NCH):
            n = c + 2
            if n < NCH:
                if n >= NB:
                    s[n - NB].wait()
                g[n] = gather(n)
            g[c].wait()
            s[c] = scatter(c)
        for c in range(NCH - NB, NCH):
            s[c].wait()

    return _sc_gather


def kernel(x, duration, max_mel_len):
    table = jnp.concatenate(
        [x.reshape(B * T, C), jnp.zeros((TPAD - B * T, C), x.dtype)], axis=0)
    buf = _idx_call(duration.astype(jnp.int32).reshape(B, 1, T)).reshape(B, MM)
    gidx = buf[:, :M].reshape(ROWS)
    total = buf[:, M]
    out = _make_sc_gather()(table, gidx)
    return out.reshape(B, M, C), total


# speedup vs baseline: 15.5765x; 15.5431x over previous
import jax
import jax.numpy as jnp
from jax import lax
from jax.experimental import pallas as pl
from jax.experimental.pallas import tpu as pltpu

B, T, C, M = 16, 512, 256, 2048
BM = 512
NMB = M // BM


def _body(dur_ref, x_ref, out_ref, len_ref, tri_ref, mf_ref, cs_ref,
          cs16_ref, st16_ref, xb_ref):
    b = pl.program_id(0)
    mb = pl.program_id(1)

    @pl.when(jnp.logical_and(b == 0, mb == 0))
    def _init():
        it = lax.broadcasted_iota(jnp.int32, (T, T), 0)
        jt = lax.broadcasted_iota(jnp.int32, (T, T), 1)
        tri_ref[...] = (it <= jt).astype(jnp.float32)
        mi = lax.broadcasted_iota(jnp.int32, (BM, T), 0)
        mf_ref[...] = mi.astype(jnp.int16)

    @pl.when(mb == 0)
    def _per_batch():
        d = dur_ref[...].reshape(1, T)
        df = d.astype(jnp.float32)
        cs = jnp.dot(df, tri_ref[...], preferred_element_type=jnp.float32)
        cs_ref[...] = cs
        csi = cs.astype(jnp.int16)
        cs16_ref[...] = csi
        st16_ref[...] = csi - d.astype(jnp.int16)
        len_ref[...] = jnp.sum(d).reshape(1, 1, 1)
        xb_ref[...] = x_ref[0].astype(jnp.bfloat16)

    total = cs_ref[0, T - 1]
    base = (mb * BM).astype(jnp.float32)

    @pl.when(base < total)
    def _expand():
        mm = mf_ref[...] + (mb * BM).astype(jnp.int16)
        one = jnp.bfloat16(1)
        zero = jnp.bfloat16(0)
        csb = jnp.broadcast_to(cs16_ref[...], (BM, T))
        stb = jnp.broadcast_to(st16_ref[...], (BM, T))
        ohb = (jnp.where(mm < csb, one, zero)
               - jnp.where(mm < stb, one, zero))
        out_ref[0] = jnp.dot(ohb, xb_ref[...], preferred_element_type=jnp.float32)

    @pl.when(base >= total)
    def _zeros():
        out_ref[0] = jnp.zeros((BM, C), jnp.float32)


_call = pl.pallas_call(
    _body,
    grid=(B, NMB),
    in_specs=[
        pl.BlockSpec((1, 1, T), lambda b, mb: (b, 0, 0)),
        pl.BlockSpec((1, T, C), lambda b, mb: (b, 0, 0)),
    ],
    out_specs=[
        pl.BlockSpec((1, BM, C), lambda b, mb: (b, mb, 0)),
        pl.BlockSpec((1, 1, 1), lambda b, mb: (b, 0, 0)),
    ],
    out_shape=[
        jax.ShapeDtypeStruct((B, M, C), jnp.float32),
        jax.ShapeDtypeStruct((B, 1, 1), jnp.int32),
    ],
    scratch_shapes=[
        pltpu.VMEM((T, T), jnp.float32),
        pltpu.VMEM((BM, T), jnp.int16),
        pltpu.VMEM((1, T), jnp.float32),
        pltpu.VMEM((1, T), jnp.int16),
        pltpu.VMEM((1, T), jnp.int16),
        pltpu.VMEM((T, C), jnp.bfloat16),
    ],
)


def kernel(x, duration, max_mel_len):
    out, tot = _call(duration.astype(jnp.int32).reshape(B, 1, T), x)
    return out, tot.reshape(B)


# speedup vs baseline: 31.6844x; 2.0341x over previous
import jax
import jax.numpy as jnp
from jax import lax
from jax.experimental import pallas as pl
from jax.experimental.pallas import tpu as pltpu

B, T, C, M = 16, 512, 256, 2048
BM = 512
NMB = M // BM


def _body(dur_ref, x_ref, out_ref, len_ref, tri_ref, mf_ref,
          cs_ref, cs16_ref, st16_ref, xb_ref):
    b = pl.program_id(0)

    @pl.when(b == 0)
    def _init():
        it = lax.broadcasted_iota(jnp.int32, (T, T), 0)
        jt = lax.broadcasted_iota(jnp.int32, (T, T), 1)
        tri_ref[...] = (it <= jt).astype(jnp.float32)
        mi = lax.broadcasted_iota(jnp.int32, (BM, T), 0)
        mf_ref[...] = mi.astype(jnp.int16)

    d = dur_ref[...].reshape(1, T)
    df = d.astype(jnp.float32)
    cs = jnp.dot(df, tri_ref[...], preferred_element_type=jnp.float32)
    cs_ref[...] = cs
    csi = cs.astype(jnp.int16)
    cs16_ref[...] = csi
    st16_ref[...] = csi - d.astype(jnp.int16)
    len_ref[...] = jnp.sum(d).reshape(1, 1, 1)
    xb_ref[...] = x_ref[0].astype(jnp.bfloat16)
    total = cs_ref[0, T - 1]

    one = jnp.bfloat16(1)
    zero = jnp.bfloat16(0)
    csb = jnp.broadcast_to(cs16_ref[...], (BM, T))
    stb = jnp.broadcast_to(st16_ref[...], (BM, T))

    for sub in range(NMB):
        base = sub * BM

        @pl.when(jnp.float32(base) < total)
        def _expand(base=base):
            mm = mf_ref[...] + jnp.int16(base)
            ohb = (jnp.where(mm < csb, one, zero)
                   - jnp.where(mm < stb, one, zero))
            out_ref[0, base:base + BM, :] = jnp.dot(
                ohb, xb_ref[...], preferred_element_type=jnp.float32)

        @pl.when(jnp.float32(base) >= total)
        def _zeros(base=base):
            out_ref[0, base:base + BM, :] = jnp.zeros((BM, C), jnp.float32)


_call = pl.pallas_call(
    _body,
    grid=(B,),
    in_specs=[
        pl.BlockSpec((1, 1, T), lambda b: (b, 0, 0)),
        pl.BlockSpec((1, T, C), lambda b: (b, 0, 0)),
    ],
    out_specs=[
        pl.BlockSpec((1, M, C), lambda b: (b, 0, 0)),
        pl.BlockSpec((1, 1, 1), lambda b: (b, 0, 0)),
    ],
    out_shape=[
        jax.ShapeDtypeStruct((B, M, C), jnp.float32),
        jax.ShapeDtypeStruct((B, 1, 1), jnp.int32),
    ],
    scratch_shapes=[
        pltpu.VMEM((T, T), jnp.float32),
        pltpu.VMEM((BM, T), jnp.int16),
        pltpu.VMEM((1, T), jnp.float32),
        pltpu.VMEM((1, T), jnp.int16),
        pltpu.VMEM((1, T), jnp.int16),
        pltpu.VMEM((T, C), jnp.bfloat16),
    ],
)


def kernel(x, duration, max_mel_len):
    out, tot = _call(duration.astype(jnp.int32).reshape(B, 1, T), x)
    return out, tot.reshape(B)


# speedup vs baseline: 33.2324x; 1.0489x over previous
import jax
import jax.numpy as jnp
from jax import lax
from jax.experimental import pallas as pl
from jax.experimental.pallas import tpu as pltpu

B, T, C, M = 16, 512, 256, 2048
BM = 512
NMB = M // BM


def _body(dur_ref, x_ref, out_ref, len_ref, tri_ref, mf_ref,
          cs_ref, cs16_ref, st16_ref, xb_ref):
    b = pl.program_id(0)

    @pl.when(b == 0)
    def _init():
        it = lax.broadcasted_iota(jnp.int32, (T, T), 0)
        jt = lax.broadcasted_iota(jnp.int32, (T, T), 1)
        tri_ref[...] = (it <= jt).astype(jnp.float32)
        mi = lax.broadcasted_iota(jnp.int32, (BM, T), 0)
        mf_ref[...] = mi.astype(jnp.int16)
        d_all = dur_ref[...].reshape(B, T)
        df_all = d_all.astype(jnp.float32)
        cs_all = jnp.dot(df_all, tri_ref[...],
                         preferred_element_type=jnp.float32)
        cs_ref[:, 0:1, :] = cs_all.reshape(B, 1, T)
        csi_all = cs_all.astype(jnp.int16)
        cs16_ref[:, 0:1, :] = csi_all.reshape(B, 1, T)
        st16_ref[:, 0:1, :] = (csi_all - d_all.astype(jnp.int16)).reshape(B, 1, T)

    total = cs_ref[b, 0, T - 1]
    len_ref[...] = total.astype(jnp.int32).reshape(1, 1, 1)
    xb_ref[...] = x_ref[0].astype(jnp.bfloat16)

    one = jnp.bfloat16(1)
    zero = jnp.bfloat16(0)
    csb = jnp.broadcast_to(cs16_ref[b, 0:1, :], (BM, T))
    stb = jnp.broadcast_to(st16_ref[b, 0:1, :], (BM, T))

    for sub in range(NMB):
        base = sub * BM

        @pl.when(jnp.float32(base) < total)
        def _expand(base=base):
            mm = mf_ref[...] + jnp.int16(base)
            ohb = (jnp.where(mm < csb, one, zero)
                   - jnp.where(mm < stb, one, zero))
            out_ref[0, base:base + BM, :] = jnp.dot(
                ohb, xb_ref[...], preferred_element_type=jnp.float32)

        @pl.when(jnp.float32(base) >= total)
        def _zeros(base=base):
            out_ref[0, base:base + BM, :] = jnp.zeros((BM, C), jnp.float32)


_call = pl.pallas_call(
    _body,
    grid=(B,),
    in_specs=[
        pl.BlockSpec((B, 1, T), lambda b: (0, 0, 0)),
        pl.BlockSpec((1, T, C), lambda b: (b, 0, 0)),
    ],
    out_specs=[
        pl.BlockSpec((1, M, C), lambda b: (b, 0, 0)),
        pl.BlockSpec((1, 1, 1), lambda b: (b, 0, 0)),
    ],
    out_shape=[
        jax.ShapeDtypeStruct((B, M, C), jnp.float32),
        jax.ShapeDtypeStruct((B, 1, 1), jnp.int32),
    ],
    scratch_shapes=[
        pltpu.VMEM((T, T), jnp.float32),
        pltpu.VMEM((BM, T), jnp.int16),
        pltpu.VMEM((B, 8, T), jnp.float32),
        pltpu.VMEM((B, 16, T), jnp.int16),
        pltpu.VMEM((B, 16, T), jnp.int16),
        pltpu.VMEM((T, C), jnp.bfloat16),
    ],
)


def kernel(x, duration, max_mel_len):
    out, tot = _call(duration.astype(jnp.int32).reshape(B, 1, T), x)
    return out, tot.reshape(B)
